# Initial kernel scaffold; baseline (speedup 1.0000x reference)
import jax
import jax.numpy as jnp
from jax.experimental import pallas as pl


def kernel(atoms, deg_slice, membership, deg_adj_1, deg_adj_2, deg_adj_3, deg_adj_4, deg_adj_5, deg_adj_6, deg_adj_7, deg_adj_8, deg_adj_9, deg_adj_10):
    raise NotImplementedError("write your pallas kernel here")


# speedup vs baseline: 1.6829x; 1.6829x over previous
import functools

import jax
import jax.numpy as jnp
from jax import lax
from jax.experimental import pallas as pl
from jax.experimental.pallas import tpu as pltpu
from jax.experimental.pallas import tpu_sc as plsc

N = 100000
D = 128
CD = 9000
C0 = 10000
MAX_DEG = 10
B = 24
B0 = 40
NCH = CD // B
NCH0 = C0 // B0
GMAX = 120
EMAX = B * (MAX_DEG + 1)

_EBASE = [0] * (MAX_DEG + 1)
for _d in range(2, MAX_DEG + 1):
    _EBASE[_d] = _EBASE[_d - 1] + CD * _d


def _build_edges(deg_adj_lists):
    parts = []
    for d in range(1, MAX_DEG + 1):
        start = C0 + CD * (d - 1)
        self_idx = (start + jnp.arange(CD, dtype=jnp.int32))[:, None]
        aug = jnp.concatenate([self_idx, deg_adj_lists[d - 1]], axis=1)
        parts.append(aug.reshape(-1))
    return jnp.concatenate(parts)


def _pool(atoms, edges):
    mesh = plsc.VectorSubcoreMesh(core_axis_name="c", subcore_axis_name="s")
    nw = mesh.num_cores * mesh.num_subcores

    @functools.partial(
        pl.kernel,
        out_type=jax.ShapeDtypeStruct((N, D), jnp.float32),
        mesh=mesh,
        scratch_types=[
            pltpu.VMEM((EMAX,), jnp.int32),
            pltpu.VMEM((EMAX, D), jnp.float32),
            pltpu.VMEM((B, D), jnp.float32),
            pltpu.VMEM((B0, D), jnp.float32),
            pltpu.SemaphoreType.DMA,
        ],
    )
    def k(atoms_hbm, edges_hbm, out_hbm, idx_v, gath_v, outb_v, cpy_v, sem):
        wid = lax.axis_index("s") * mesh.num_cores + lax.axis_index("c")

        def copy_body(t, _):
            c = wid + t * nw
            r0 = c * B0
            pltpu.sync_copy(atoms_hbm.at[pl.ds(r0, B0)], cpy_v)
            pltpu.sync_copy(cpy_v, out_hbm.at[pl.ds(r0, B0)])
            return 0

        n0 = (NCH0 - wid + nw - 1) // nw
        lax.fori_loop(0, n0, copy_body, 0)

        for d in range(1, MAX_DEG + 1):
            m = d + 1
            ecount = B * m
            ebase = _EBASE[d]
            start = C0 + CD * (d - 1)
            pieces = [(off, min(GMAX, ecount - off))
                      for off in range(0, ecount, GMAX)]

            def chunk_body(t, _, m=m, ecount=ecount, ebase=ebase,
                           start=start, pieces=pieces):
                c = wid + t * nw
                pltpu.sync_copy(edges_hbm.at[pl.ds(ebase + c * ecount, ecount)],
                                idx_v.at[pl.ds(0, ecount)])
                cps = [pltpu.async_copy(atoms_hbm.at[idx_v.at[pl.ds(off, sz)]],
                                        gath_v.at[pl.ds(off, sz)], sem)
                       for off, sz in pieces]
                for cp in cps:
                    cp.wait()

                def row_body(i, _):
                    base = i * m
                    for s in range(D // 16):
                        sl = pl.ds(s * 16, 16)
                        acc = gath_v[base, sl]
                        for j in range(1, m):
                            acc = jnp.maximum(acc, gath_v[base + j, sl])
                        outb_v[i, sl] = acc
                    return 0

                lax.fori_loop(0, B, row_body, 0)
                pltpu.sync_copy(outb_v, out_hbm.at[pl.ds(start + c * B, B)])
                return 0

            nch_w = (NCH - wid + nw - 1) // nw
            lax.fori_loop(0, nch_w, chunk_body, 0)

    return k(atoms, edges)


def kernel(atoms, deg_slice, membership, deg_adj_1, deg_adj_2, deg_adj_3,
           deg_adj_4, deg_adj_5, deg_adj_6, deg_adj_7, deg_adj_8,
           deg_adj_9, deg_adj_10):
    del deg_slice, membership
    edges = _build_edges([deg_adj_1, deg_adj_2, deg_adj_3, deg_adj_4,
                          deg_adj_5, deg_adj_6, deg_adj_7, deg_adj_8,
                          deg_adj_9, deg_adj_10])
    return _pool(atoms, edges)


# speedup vs baseline: 2.4286x; 1.4431x over previous
import functools

import jax
import jax.numpy as jnp
from jax import lax
from jax.experimental import pallas as pl
from jax.experimental.pallas import tpu as pltpu
from jax.experimental.pallas import tpu_sc as plsc

N = 100000
D = 128
CD = 9000
C0 = 10000
MAX_DEG = 10
B = 24
B0 = 40
NCH = CD // B
NCH0 = C0 // B0
GMAX = 120
EMAX = B * (MAX_DEG + 1)

_EBASE = [0] * (MAX_DEG + 1)
for _d in range(2, MAX_DEG + 1):
    _EBASE[_d] = _EBASE[_d - 1] + CD * _d


def _build_edges(deg_adj_lists):
    parts = []
    for d in range(1, MAX_DEG + 1):
        start = C0 + CD * (d - 1)
        self_idx = (start + jnp.arange(CD, dtype=jnp.int32))[:, None]
        aug = jnp.concatenate([self_idx, deg_adj_lists[d - 1]], axis=1)
        parts.append(aug.reshape(-1))
    return jnp.concatenate(parts)


def _pool(atoms, edges):
    mesh = plsc.VectorSubcoreMesh(core_axis_name="c", subcore_axis_name="s")
    nw = mesh.num_cores * mesh.num_subcores

    @functools.partial(
        pl.kernel,
        out_type=jax.ShapeDtypeStruct((N, D), jnp.float32),
        mesh=mesh,
        scratch_types=[
            pltpu.VMEM((2 * EMAX,), jnp.int32),
            pltpu.VMEM((2, EMAX, D), jnp.float32),
            pltpu.VMEM((2, B, D), jnp.float32),
            pltpu.VMEM((B0, D), jnp.float32),
            pltpu.SemaphoreType.DMA,
            pltpu.SemaphoreType.DMA,
            pltpu.SemaphoreType.DMA,
            pltpu.SemaphoreType.DMA,
        ],
    )
    def k(atoms_hbm, edges_hbm, out_hbm, idx_v, gath_v, outb_v, cpy_v,
          gsem0, gsem1, ssem0, ssem1):
        gsems = (gsem0, gsem1)
        ssems = (ssem0, ssem1)
        wid = lax.axis_index("s") * mesh.num_cores + lax.axis_index("c")

        @pl.loop(0, (NCH0 - wid + nw - 1) // nw)
        def copy_body(t):
            r0 = (wid + t * nw) * B0
            pltpu.sync_copy(atoms_hbm.at[pl.ds(r0, B0)], cpy_v)
            pltpu.sync_copy(cpy_v, out_hbm.at[pl.ds(r0, B0)])

        for d in range(1, MAX_DEG + 1):
            m = d + 1
            ecount = B * m
            ebase = _EBASE[d]
            start = C0 + CD * (d - 1)
            pieces = [(off, min(GMAX, ecount - off))
                      for off in range(0, ecount, GMAX)]
            nch_w = (NCH - wid + nw - 1) // nw

            def issue(kk, b, ecount=ecount, ebase=ebase, pieces=pieces):
                c = wid + kk * nw
                pltpu.sync_copy(
                    edges_hbm.at[pl.ds(ebase + c * ecount, ecount)],
                    idx_v.at[pl.ds(b * EMAX, ecount)])
                for off, sz in pieces:
                    pltpu.async_copy(
                        atoms_hbm.at[idx_v.at[pl.ds(b * EMAX + off, sz)]],
                        gath_v.at[b, pl.ds(off, sz)], gsems[b])

            def compute(kk, b, m=m, ecount=ecount, start=start):
                c = wid + kk * nw
                pltpu.make_async_copy(
                    atoms_hbm.at[pl.ds(0, ecount)],
                    gath_v.at[b, pl.ds(0, ecount)], gsems[b]).wait()

                @pl.when(kk >= 2)
                def _():
                    pltpu.make_async_copy(
                        outb_v.at[b], out_hbm.at[pl.ds(0, B)], ssems[b]).wait()

                @pl.loop(0, B)
                def row_body(i):
                    base = i * m
                    for s in range(D // 16):
                        sl = pl.ds(s * 16, 16)
                        acc = gath_v[b, base, sl]
                        for j in range(1, m):
                            acc = jnp.maximum(acc, gath_v[b, base + j, sl])
                        outb_v[b, i, sl] = acc

                pltpu.async_copy(outb_v.at[b],
                                 out_hbm.at[pl.ds(start + c * B, B)], ssems[b])

            issue(0, 0)
            issue(1, 1)

            @pl.loop(0, nch_w, step=2)
            def chunk_body(t, nch_w=nch_w, issue=issue, compute=compute):
                for b in (0, 1):
                    kk = t + b

                    @pl.when(kk < nch_w)
                    def _(kk=kk, b=b):
                        compute(kk, b)

                        @pl.when(kk + 2 < nch_w)
                        def __():
                            issue(kk + 2, b)

            for b in (0, 1):
                pltpu.make_async_copy(
                    outb_v.at[b], out_hbm.at[pl.ds(0, B)], ssems[b]).wait()

    return k(atoms, edges)


def kernel(atoms, deg_slice, membership, deg_adj_1, deg_adj_2, deg_adj_3,
           deg_adj_4, deg_adj_5, deg_adj_6, deg_adj_7, deg_adj_8,
           deg_adj_9, deg_adj_10):
    del deg_slice, membership
    edges = _build_edges([deg_adj_1, deg_adj_2, deg_adj_3, deg_adj_4,
                          deg_adj_5, deg_adj_6, deg_adj_7, deg_adj_8,
                          deg_adj_9, deg_adj_10])
    return _pool(atoms, edges)


# speedup vs baseline: 2.7780x; 1.1439x over previous
import functools

import jax
import jax.numpy as jnp
from jax import lax
from jax.experimental import pallas as pl
from jax.experimental.pallas import tpu as pltpu
from jax.experimental.pallas import tpu_sc as plsc

N = 100000
D = 128
CD = 9000
C0 = 10000
MAX_DEG = 10
B = 24
NCH = CD // B
GMAX = 120
EMAX = B * (MAX_DEG + 1)

_EBASE = [0] * (MAX_DEG + 1)
for _d in range(2, MAX_DEG + 1):
    _EBASE[_d] = _EBASE[_d - 1] + CD * _d


def _build_edges(deg_adj_lists):
    parts = []
    for d in range(1, MAX_DEG + 1):
        start = C0 + CD * (d - 1)
        self_idx = (start + jnp.arange(CD, dtype=jnp.int32))[:, None]
        aug = jnp.concatenate([self_idx, deg_adj_lists[d - 1]], axis=1)
        parts.append(aug.reshape(-1))
    return jnp.concatenate(parts)


def _pool(atoms, edges):
    mesh = plsc.VectorSubcoreMesh(core_axis_name="c", subcore_axis_name="s")
    nw = mesh.num_cores * mesh.num_subcores

    @functools.partial(
        pl.kernel,
        out_type=jax.ShapeDtypeStruct((N, D), jnp.float32),
        mesh=mesh,
        scratch_types=[
            pltpu.VMEM((2 * EMAX,), jnp.int32),
            pltpu.VMEM((2, EMAX, D), jnp.float32),
            pltpu.VMEM((2, B, D), jnp.float32),
            pltpu.SemaphoreType.DMA,
            pltpu.SemaphoreType.DMA,
            pltpu.SemaphoreType.DMA,
            pltpu.SemaphoreType.DMA,
            pltpu.SemaphoreType.DMA,
            pltpu.SemaphoreType.DMA,
            pltpu.SemaphoreType.DMA,
        ],
    )
    def k(atoms_hbm, edges_hbm, out_hbm, idx_v, gath_v, outb_v,
          gsem0, gsem1, isem0, isem1, ssem0, ssem1, dsem):
        gsems = (gsem0, gsem1)
        isems = (isem0, isem1)
        ssems = (ssem0, ssem1)
        wid = lax.axis_index("s") * mesh.num_cores + lax.axis_index("c")

        r0a = pl.multiple_of(
            312 * wid + 8 * jnp.minimum(wid, 2), 8)

        @pl.when(wid < 2)
        def _():
            pltpu.async_copy(atoms_hbm.at[pl.ds(r0a, 320)],
                             out_hbm.at[pl.ds(r0a, 320)], dsem)

        @pl.when(wid >= 2)
        def _():
            pltpu.async_copy(atoms_hbm.at[pl.ds(r0a, 312)],
                             out_hbm.at[pl.ds(r0a, 312)], dsem)

        for d in range(1, MAX_DEG + 1):
            m = d + 1
            ecount = B * m
            ebase = _EBASE[d]
            start = C0 + CD * (d - 1)
            pieces = [(off, min(GMAX, ecount - off))
                      for off in range(0, ecount, GMAX)]
            rot = (13 * d) % nw
            cw = (wid + rot) % nw
            nch_w = (NCH - cw + nw - 1) // nw

            def issue_idx(kk, b, ecount=ecount, ebase=ebase, cw=cw):
                c = cw + kk * nw
                pltpu.async_copy(
                    edges_hbm.at[pl.ds(ebase + c * ecount, ecount)],
                    idx_v.at[pl.ds(b * EMAX, ecount)], isems[b])

            def wait_idx(b, ecount=ecount):
                pltpu.make_async_copy(
                    edges_hbm.at[pl.ds(0, ecount)],
                    idx_v.at[pl.ds(b * EMAX, ecount)], isems[b]).wait()

            def issue_gath(b, pieces=pieces):
                for off, sz in pieces:
                    pltpu.async_copy(
                        atoms_hbm.at[idx_v.at[pl.ds(b * EMAX + off, sz)]],
                        gath_v.at[b, pl.ds(off, sz)], gsems[b])

            def wait_gath(b, ecount=ecount):
                pltpu.make_async_copy(
                    atoms_hbm.at[pl.ds(0, ecount)],
                    gath_v.at[b, pl.ds(0, ecount)], gsems[b]).wait()

            def compute(kk, b, m=m, start=start, cw=cw):
                c = cw + kk * nw

                @pl.when(kk >= 2)
                def _():
                    pltpu.make_async_copy(
                        outb_v.at[b], out_hbm.at[pl.ds(0, B)], ssems[b]).wait()

                @pl.loop(0, B)
                def row_body(i):
                    base = i * m
                    for s in range(D // 16):
                        sl = pl.ds(s * 16, 16)
                        acc = gath_v[b, base, sl]
                        for j in range(1, m):
                            acc = jnp.maximum(acc, gath_v[b, base + j, sl])
                        outb_v[b, i, sl] = acc

                pltpu.async_copy(outb_v.at[b],
                                 out_hbm.at[pl.ds(start + c * B, B)], ssems[b])

            issue_idx(0, 0)
            issue_idx(1, 1)
            wait_idx(0)
            issue_gath(0)

            @pl.loop(0, nch_w)
            def chunk_body(kk, nch_w=nch_w, issue_idx=issue_idx,
                           wait_idx=wait_idx, issue_gath=issue_gath,
                           wait_gath=wait_gath, compute=compute):
                for b in (0, 1):
                    @pl.when(kk % 2 == b)
                    def _(b=b, kk=kk):
                        wait_gath(b)

                        @pl.when(kk + 2 < nch_w)
                        def _():
                            issue_idx(kk + 2, b)

                        @pl.when(kk + 1 < nch_w)
                        def _(b=b):
                            wait_idx(1 - b)
                            issue_gath(1 - b)

                        compute(kk, b)

            for b in (0, 1):
                pltpu.make_async_copy(
                    outb_v.at[b], out_hbm.at[pl.ds(0, B)], ssems[b]).wait()

        @pl.when(wid < 2)
        def _():
            pltpu.make_async_copy(atoms_hbm.at[pl.ds(0, 320)],
                                  out_hbm.at[pl.ds(0, 320)], dsem).wait()

        @pl.when(wid >= 2)
        def _():
            pltpu.make_async_copy(atoms_hbm.at[pl.ds(0, 312)],
                                  out_hbm.at[pl.ds(0, 312)], dsem).wait()

    return k(atoms, edges)


def kernel(atoms, deg_slice, membership, deg_adj_1, deg_adj_2, deg_adj_3,
           deg_adj_4, deg_adj_5, deg_adj_6, deg_adj_7, deg_adj_8,
           deg_adj_9, deg_adj_10):
    del deg_slice, membership
    edges = _build_edges([deg_adj_1, deg_adj_2, deg_adj_3, deg_adj_4,
                          deg_adj_5, deg_adj_6, deg_adj_7, deg_adj_8,
                          deg_adj_9, deg_adj_10])
    return _pool(atoms, edges)


# speedup vs baseline: 3.4236x; 1.2324x over previous
import functools

import jax
import jax.numpy as jnp
from jax import lax
from jax.experimental import pallas as pl
from jax.experimental.pallas import tpu as pltpu
from jax.experimental.pallas import tpu_sc as plsc

N = 100000
D = 128
CD = 9000
C0 = 10000
MAX_DEG = 10
B = 24
NCH = CD // B
GMAX = 120
EMAX = B * MAX_DEG


def _pool(atoms, adj_flats):
    mesh = plsc.VectorSubcoreMesh(core_axis_name="c", subcore_axis_name="s")
    nw = mesh.num_cores * mesh.num_subcores

    @functools.partial(
        pl.kernel,
        out_type=jax.ShapeDtypeStruct((N, D), jnp.float32),
        mesh=mesh,
        scratch_types=[
            pltpu.VMEM((2 * EMAX,), jnp.int32),
            pltpu.VMEM((2, EMAX, D), jnp.float32),
            pltpu.VMEM((2, B, D), jnp.float32),
            pltpu.VMEM((2, B, D), jnp.float32),
            pltpu.SemaphoreType.DMA,
            pltpu.SemaphoreType.DMA,
            pltpu.SemaphoreType.DMA,
            pltpu.SemaphoreType.DMA,
            pltpu.SemaphoreType.DMA,
            pltpu.SemaphoreType.DMA,
            pltpu.SemaphoreType.DMA,
            pltpu.SemaphoreType.DMA,
            pltpu.SemaphoreType.DMA,
        ],
    )
    def k(atoms_hbm, a1, a2, a3, a4, a5, a6, a7, a8, a9, a10, out_hbm,
          idx_v, gath_v, self_v, outb_v,
          gsem0, gsem1, isem0, isem1, fsem0, fsem1, ssem0, ssem1, dsem):
        adjs = (a1, a2, a3, a4, a5, a6, a7, a8, a9, a10)
        gsems = (gsem0, gsem1)
        isems = (isem0, isem1)
        fsems = (fsem0, fsem1)
        ssems = (ssem0, ssem1)
        wid = lax.axis_index("s") * mesh.num_cores + lax.axis_index("c")

        r0a = pl.multiple_of(312 * wid + 8 * jnp.minimum(wid, 2), 8)

        @pl.when(wid < 2)
        def _():
            pltpu.async_copy(atoms_hbm.at[pl.ds(r0a, 320)],
                             out_hbm.at[pl.ds(r0a, 320)], dsem)

        @pl.when(wid >= 2)
        def _():
            pltpu.async_copy(atoms_hbm.at[pl.ds(r0a, 312)],
                             out_hbm.at[pl.ds(r0a, 312)], dsem)

        for d in range(1, MAX_DEG + 1):
            ecount = B * d
            edges_hbm = adjs[d - 1]
            start = C0 + CD * (d - 1)
            pieces = [(off, min(GMAX, ecount - off))
                      for off in range(0, ecount, GMAX)]
            rot = (13 * d) % nw
            cw = (wid + rot) % nw
            nch_w = (NCH - cw + nw - 1) // nw

            def issue_idx(kk, b, ecount=ecount, edges_hbm=edges_hbm, cw=cw):
                c = cw + kk * nw
                pltpu.async_copy(
                    edges_hbm.at[pl.ds(c * ecount, ecount)],
                    idx_v.at[pl.ds(b * EMAX, ecount)], isems[b])

            def wait_idx(b, ecount=ecount, edges_hbm=edges_hbm):
                pltpu.make_async_copy(
                    edges_hbm.at[pl.ds(0, ecount)],
                    idx_v.at[pl.ds(b * EMAX, ecount)], isems[b]).wait()

            def issue_self(kk, b, start=start, cw=cw):
                c = cw + kk * nw
                pltpu.async_copy(atoms_hbm.at[pl.ds(start + c * B, B)],
                                 self_v.at[b], fsems[b])

            def wait_self(b):
                pltpu.make_async_copy(atoms_hbm.at[pl.ds(0, B)],
                                      self_v.at[b], fsems[b]).wait()

            def issue_gath(b, pieces=pieces):
                for off, sz in pieces:
                    pltpu.async_copy(
                        atoms_hbm.at[idx_v.at[pl.ds(b * EMAX + off, sz)]],
                        gath_v.at[b, pl.ds(off, sz)], gsems[b])

            def wait_gath(b, ecount=ecount):
                pltpu.make_async_copy(
                    atoms_hbm.at[pl.ds(0, ecount)],
                    gath_v.at[b, pl.ds(0, ecount)], gsems[b]).wait()

            def compute(kk, b, d=d, start=start, cw=cw):
                c = cw + kk * nw

                @pl.when(kk >= 2)
                def _():
                    pltpu.make_async_copy(
                        outb_v.at[b], out_hbm.at[pl.ds(0, B)], ssems[b]).wait()

                wait_self(b)

                @pl.loop(0, B)
                def row_body(i):
                    base = i * d
                    for s in range(D // 16):
                        sl = pl.ds(s * 16, 16)
                        acc = self_v[b, i, sl]
                        for j in range(d):
                            acc = jnp.maximum(acc, gath_v[b, base + j, sl])
                        outb_v[b, i, sl] = acc

                pltpu.async_copy(outb_v.at[b],
                                 out_hbm.at[pl.ds(start + c * B, B)], ssems[b])

            issue_idx(0, 0)
            issue_idx(1, 1)
            issue_self(0, 0)
            issue_self(1, 1)
            wait_idx(0)
            issue_gath(0)

            @pl.loop(0, nch_w)
            def chunk_body(kk, nch_w=nch_w, issue_idx=issue_idx,
                           wait_idx=wait_idx, issue_gath=issue_gath,
                           wait_gath=wait_gath, issue_self=issue_self,
                           compute=compute):
                for b in (0, 1):
                    @pl.when(kk % 2 == b)
                    def _(b=b, kk=kk):
                        wait_gath(b)

                        @pl.when(kk + 2 < nch_w)
                        def _():
                            issue_idx(kk + 2, b)

                        @pl.when(kk + 1 < nch_w)
                        def _(b=b):
                            wait_idx(1 - b)
                            issue_gath(1 - b)

                        compute(kk, b)

                        @pl.when(kk + 2 < nch_w)
                        def _():
                            issue_self(kk + 2, b)

            for b in (0, 1):
                pltpu.make_async_copy(
                    outb_v.at[b], out_hbm.at[pl.ds(0, B)], ssems[b]).wait()

        @pl.when(wid < 2)
        def _():
            pltpu.make_async_copy(atoms_hbm.at[pl.ds(0, 320)],
                                  out_hbm.at[pl.ds(0, 320)], dsem).wait()

        @pl.when(wid >= 2)
        def _():
            pltpu.make_async_copy(atoms_hbm.at[pl.ds(0, 312)],
                                  out_hbm.at[pl.ds(0, 312)], dsem).wait()

    return k(atoms, *adj_flats)


def kernel(atoms, deg_slice, membership, deg_adj_1, deg_adj_2, deg_adj_3,
           deg_adj_4, deg_adj_5, deg_adj_6, deg_adj_7, deg_adj_8,
           deg_adj_9, deg_adj_10):
    del deg_slice, membership
    adj_flats = [a.reshape(-1) for a in
                 (deg_adj_1, deg_adj_2, deg_adj_3, deg_adj_4, deg_adj_5,
                  deg_adj_6, deg_adj_7, deg_adj_8, deg_adj_9, deg_adj_10)]
    return _pool(atoms, adj_flats)


# speedup vs baseline: 3.5870x; 1.0477x over previous
import functools

import jax
import jax.numpy as jnp
from jax import lax
from jax.experimental import pallas as pl
from jax.experimental.pallas import tpu as pltpu
from jax.experimental.pallas import tpu_sc as plsc

N = 100000
D = 128
CD = 9000
C0 = 10000
MAX_DEG = 10
B = 40
NCH = CD // B
GMAX = 120
EMAX = B * MAX_DEG


def _pool(atoms, adj_flats):
    mesh = plsc.VectorSubcoreMesh(core_axis_name="c", subcore_axis_name="s")
    nw = mesh.num_cores * mesh.num_subcores

    @functools.partial(
        pl.kernel,
        out_type=jax.ShapeDtypeStruct((N, D), jnp.float32),
        mesh=mesh,
        scratch_types=[
            pltpu.VMEM((2 * EMAX,), jnp.int32),
            pltpu.VMEM((2, EMAX, D), jnp.float32),
            pltpu.VMEM((2, B, D), jnp.float32),
            pltpu.VMEM((2, B, D), jnp.float32),
            pltpu.SemaphoreType.DMA,
            pltpu.SemaphoreType.DMA,
            pltpu.SemaphoreType.DMA,
            pltpu.SemaphoreType.DMA,
            pltpu.SemaphoreType.DMA,
            pltpu.SemaphoreType.DMA,
            pltpu.SemaphoreType.DMA,
            pltpu.SemaphoreType.DMA,
            pltpu.SemaphoreType.DMA,
        ],
    )
    def k(atoms_hbm, a1, a2, a3, a4, a5, a6, a7, a8, a9, a10, out_hbm,
          idx_v, gath_v, self_v, outb_v,
          gsem0, gsem1, isem0, isem1, fsem0, fsem1, ssem0, ssem1, dsem):
        adjs = (a1, a2, a3, a4, a5, a6, a7, a8, a9, a10)
        gsems = (gsem0, gsem1)
        isems = (isem0, isem1)
        fsems = (fsem0, fsem1)
        ssems = (ssem0, ssem1)
        wid = lax.axis_index("s") * mesh.num_cores + lax.axis_index("c")

        r0a = pl.multiple_of(312 * wid + 8 * jnp.minimum(wid, 2), 8)

        @pl.when(wid < 2)
        def _():
            pltpu.async_copy(atoms_hbm.at[pl.ds(r0a, 320)],
                             out_hbm.at[pl.ds(r0a, 320)], dsem)

        @pl.when(wid >= 2)
        def _():
            pltpu.async_copy(atoms_hbm.at[pl.ds(r0a, 312)],
                             out_hbm.at[pl.ds(r0a, 312)], dsem)

        for d in range(1, MAX_DEG + 1):
            ecount = B * d
            edges_hbm = adjs[d - 1]
            start = C0 + CD * (d - 1)
            pieces = [(off, min(GMAX, ecount - off))
                      for off in range(0, ecount, GMAX)]
            rot = (13 * d) % nw
            cw = (wid + rot) % nw
            nch_w = (NCH - cw + nw - 1) // nw

            def issue_idx(kk, b, ecount=ecount, edges_hbm=edges_hbm, cw=cw):
                c = cw + kk * nw
                pltpu.async_copy(
                    edges_hbm.at[pl.ds(c * ecount, ecount)],
                    idx_v.at[pl.ds(b * EMAX, ecount)], isems[b])

            def wait_idx(b, ecount=ecount, edges_hbm=edges_hbm):
                pltpu.make_async_copy(
                    edges_hbm.at[pl.ds(0, ecount)],
                    idx_v.at[pl.ds(b * EMAX, ecount)], isems[b]).wait()

            def issue_self(kk, b, start=start, cw=cw):
                c = cw + kk * nw
                pltpu.async_copy(atoms_hbm.at[pl.ds(start + c * B, B)],
                                 self_v.at[b], fsems[b])

            def wait_self(b):
                pltpu.make_async_copy(atoms_hbm.at[pl.ds(0, B)],
                                      self_v.at[b], fsems[b]).wait()

            def issue_gath(b, pieces=pieces):
                for off, sz in pieces:
                    pltpu.async_copy(
                        atoms_hbm.at[idx_v.at[pl.ds(b * EMAX + off, sz)]],
                        gath_v.at[b, pl.ds(off, sz)], gsems[b])

            def wait_gath(b, ecount=ecount):
                pltpu.make_async_copy(
                    atoms_hbm.at[pl.ds(0, ecount)],
                    gath_v.at[b, pl.ds(0, ecount)], gsems[b]).wait()

            def compute(kk, b, d=d, start=start, cw=cw):
                c = cw + kk * nw

                @pl.when(kk >= 2)
                def _():
                    pltpu.make_async_copy(
                        outb_v.at[b], out_hbm.at[pl.ds(0, B)], ssems[b]).wait()

                wait_self(b)

                @pl.loop(0, B)
                def row_body(i):
                    base = i * d
                    for s in range(D // 16):
                        sl = pl.ds(s * 16, 16)
                        acc = self_v[b, i, sl]
                        for j in range(d):
                            acc = jnp.maximum(acc, gath_v[b, base + j, sl])
                        outb_v[b, i, sl] = acc

                pltpu.async_copy(outb_v.at[b],
                                 out_hbm.at[pl.ds(start + c * B, B)], ssems[b])

            issue_idx(0, 0)
            issue_idx(1, 1)
            issue_self(0, 0)
            issue_self(1, 1)
            wait_idx(0)
            issue_gath(0)

            @pl.loop(0, nch_w)
            def chunk_body(kk, nch_w=nch_w, issue_idx=issue_idx,
                           wait_idx=wait_idx, issue_gath=issue_gath,
                           wait_gath=wait_gath, issue_self=issue_self,
                           compute=compute):
                for b in (0, 1):
                    @pl.when(kk % 2 == b)
                    def _(b=b, kk=kk):
                        wait_gath(b)

                        @pl.when(kk + 2 < nch_w)
                        def _():
                            issue_idx(kk + 2, b)

                        @pl.when(kk + 1 < nch_w)
                        def _(b=b):
                            wait_idx(1 - b)
                            issue_gath(1 - b)

                        compute(kk, b)

                        @pl.when(kk + 2 < nch_w)
                        def _():
                            issue_self(kk + 2, b)

            for b in (0, 1):
                pltpu.make_async_copy(
                    outb_v.at[b], out_hbm.at[pl.ds(0, B)], ssems[b]).wait()

        @pl.when(wid < 2)
        def _():
            pltpu.make_async_copy(atoms_hbm.at[pl.ds(0, 320)],
                                  out_hbm.at[pl.ds(0, 320)], dsem).wait()

        @pl.when(wid >= 2)
        def _():
            pltpu.make_async_copy(atoms_hbm.at[pl.ds(0, 312)],
                                  out_hbm.at[pl.ds(0, 312)], dsem).wait()

    return k(atoms, *adj_flats)


def kernel(atoms, deg_slice, membership, deg_adj_1, deg_adj_2, deg_adj_3,
           deg_adj_4, deg_adj_5, deg_adj_6, deg_adj_7, deg_adj_8,
           deg_adj_9, deg_adj_10):
    del deg_slice, membership
    adj_flats = [a.reshape(-1) for a in
                 (deg_adj_1, deg_adj_2, deg_adj_3, deg_adj_4, deg_adj_5,
                  deg_adj_6, deg_adj_7, deg_adj_8, deg_adj_9, deg_adj_10)]
    return _pool(atoms, adj_flats)


# speedup vs baseline: 3.9704x; 1.1069x over previous
import functools

import jax
import jax.numpy as jnp
from jax import lax
from jax.experimental import pallas as pl
from jax.experimental.pallas import tpu as pltpu
from jax.experimental.pallas import tpu_sc as plsc

N = 100000
D = 128
CD = 9000
C0 = 10000
MAX_DEG = 10
B = 40
NCH = CD // B
GMAX = 120
EMAX = B * MAX_DEG


def _pool(atoms, adj_flats):
    mesh = plsc.VectorSubcoreMesh(core_axis_name="c", subcore_axis_name="s")
    nw = mesh.num_cores * mesh.num_subcores

    @functools.partial(
        pl.kernel,
        out_type=jax.ShapeDtypeStruct((N, D), jnp.float32),
        mesh=mesh,
        scratch_types=[
            pltpu.VMEM((2 * EMAX,), jnp.int32),
            pltpu.VMEM((2, EMAX, D), jnp.float32),
            pltpu.VMEM((2, B, D), jnp.float32),
            pltpu.VMEM((2, B, D), jnp.float32),
            pltpu.SemaphoreType.DMA,
            pltpu.SemaphoreType.DMA,
            pltpu.SemaphoreType.DMA,
            pltpu.SemaphoreType.DMA,
            pltpu.SemaphoreType.DMA,
            pltpu.SemaphoreType.DMA,
            pltpu.SemaphoreType.DMA,
            pltpu.SemaphoreType.DMA,
            pltpu.SemaphoreType.DMA,
        ],
    )
    def k(atoms_hbm, a1, a2, a3, a4, a5, a6, a7, a8, a9, a10, out_hbm,
          idx_v, gath_v, self_v, outb_v,
          gsem0, gsem1, isem0, isem1, fsem0, fsem1, ssem0, ssem1, dsem):
        adjs = (a1, a2, a3, a4, a5, a6, a7, a8, a9, a10)
        gsems = (gsem0, gsem1)
        isems = (isem0, isem1)
        fsems = (fsem0, fsem1)
        ssems = (ssem0, ssem1)
        wid = lax.axis_index("s") * mesh.num_cores + lax.axis_index("c")

        r0a = pl.multiple_of(312 * wid + 8 * jnp.minimum(wid, 2), 8)

        @pl.when(wid < 2)
        def _():
            pltpu.async_copy(atoms_hbm.at[pl.ds(r0a, 320)],
                             out_hbm.at[pl.ds(r0a, 320)], dsem)

        @pl.when(wid >= 2)
        def _():
            pltpu.async_copy(atoms_hbm.at[pl.ds(r0a, 312)],
                             out_hbm.at[pl.ds(r0a, 312)], dsem)

        for d in range(1, MAX_DEG + 1):
            ecount = B * d
            edges_hbm = adjs[d - 1]
            start = C0 + CD * (d - 1)
            pieces = [(off, min(GMAX, ecount - off))
                      for off in range(0, ecount, GMAX)]
            rot = (13 * d) % nw
            cw = (wid + rot) % nw
            nch_w = (NCH - cw + nw - 1) // nw

            def issue_idx(kk, b, ecount=ecount, edges_hbm=edges_hbm, cw=cw):
                c = cw + kk * nw
                pltpu.async_copy(
                    edges_hbm.at[pl.ds(c * ecount, ecount)],
                    idx_v.at[pl.ds(b * EMAX, ecount)], isems[b])

            def wait_idx(b, ecount=ecount, edges_hbm=edges_hbm):
                pltpu.make_async_copy(
                    edges_hbm.at[pl.ds(0, ecount)],
                    idx_v.at[pl.ds(b * EMAX, ecount)], isems[b]).wait()

            def issue_self(kk, b, start=start, cw=cw):
                c = cw + kk * nw
                pltpu.async_copy(atoms_hbm.at[pl.ds(start + c * B, B)],
                                 self_v.at[b], fsems[b])

            def wait_self(b):
                pltpu.make_async_copy(atoms_hbm.at[pl.ds(0, B)],
                                      self_v.at[b], fsems[b]).wait()

            def issue_gath(b, pieces=pieces):
                for off, sz in pieces:
                    pltpu.async_copy(
                        atoms_hbm.at[idx_v.at[pl.ds(b * EMAX + off, sz)]],
                        gath_v.at[b, pl.ds(off, sz)], gsems[b])

            def wait_gath(b, ecount=ecount):
                pltpu.make_async_copy(
                    atoms_hbm.at[pl.ds(0, ecount)],
                    gath_v.at[b, pl.ds(0, ecount)], gsems[b]).wait()

            def compute(kk, b, d=d, start=start, cw=cw):
                c = cw + kk * nw

                @pl.when(kk >= 2)
                def _():
                    pltpu.make_async_copy(
                        outb_v.at[b], out_hbm.at[pl.ds(0, B)], ssems[b]).wait()

                wait_self(b)

                @pl.loop(0, B)
                def row_body(i):
                    base = i * d
                    accs = [self_v[b, i, pl.ds(s * 16, 16)]
                            for s in range(D // 16)]
                    for j in range(d):
                        for s in range(D // 16):
                            accs[s] = jnp.maximum(
                                accs[s], gath_v[b, base + j, pl.ds(s * 16, 16)])
                    for s in range(D // 16):
                        outb_v[b, i, pl.ds(s * 16, 16)] = accs[s]

                pltpu.async_copy(outb_v.at[b],
                                 out_hbm.at[pl.ds(start + c * B, B)], ssems[b])

            issue_idx(0, 0)
            issue_idx(1, 1)
            issue_self(0, 0)
            issue_self(1, 1)
            wait_idx(0)
            issue_gath(0)

            @pl.loop(0, nch_w)
            def chunk_body(kk, nch_w=nch_w, issue_idx=issue_idx,
                           wait_idx=wait_idx, issue_gath=issue_gath,
                           wait_gath=wait_gath, issue_self=issue_self,
                           compute=compute):
                for b in (0, 1):
                    @pl.when(kk % 2 == b)
                    def _(b=b, kk=kk):
                        wait_gath(b)

                        @pl.when(kk + 2 < nch_w)
                        def _():
                            issue_idx(kk + 2, b)

                        @pl.when(kk + 1 < nch_w)
                        def _(b=b):
                            wait_idx(1 - b)
                            issue_gath(1 - b)

                        compute(kk, b)

                        @pl.when(kk + 2 < nch_w)
                        def _():
                            issue_self(kk + 2, b)

            for b in (0, 1):
                pltpu.make_async_copy(
                    outb_v.at[b], out_hbm.at[pl.ds(0, B)], ssems[b]).wait()

        @pl.when(wid < 2)
        def _():
            pltpu.make_async_copy(atoms_hbm.at[pl.ds(0, 320)],
                                  out_hbm.at[pl.ds(0, 320)], dsem).wait()

        @pl.when(wid >= 2)
        def _():
            pltpu.make_async_copy(atoms_hbm.at[pl.ds(0, 312)],
                                  out_hbm.at[pl.ds(0, 312)], dsem).wait()

    return k(atoms, *adj_flats)


def kernel(atoms, deg_slice, membership, deg_adj_1, deg_adj_2, deg_adj_3,
           deg_adj_4, deg_adj_5, deg_adj_6, deg_adj_7, deg_adj_8,
           deg_adj_9, deg_adj_10):
    del deg_slice, membership
    adj_flats = [a.reshape(-1) for a in
                 (deg_adj_1, deg_adj_2, deg_adj_3, deg_adj_4, deg_adj_5,
                  deg_adj_6, deg_adj_7, deg_adj_8, deg_adj_9, deg_adj_10)]
    return _pool(atoms, adj_flats)
